# baseline (speedup 1.0000x reference)
import functools

import jax
import jax.numpy as jnp
from jax import lax
from jax.experimental import pallas as pl
from jax.experimental.pallas import tpu as pltpu
from jax.experimental.pallas import tpu_sc as plsc

B = 16384
D = 64
L = 16
NC = 2
NS = 16
NW = NC * NS
BPW = B // NW
CHUNK = 128
NCHUNK = BPW // CHUNK
DP = 128

_SC_MESH = dict(
    mesh=plsc.VectorSubcoreMesh(core_axis_name="c", subcore_axis_name="s"),
)


def _dot_body(u_hbm, i_hbm, p_hbm, q_hbm, out_hbm,
              uidx_v, iidx_v, uoff_v, ioff_v,
              prow_v, qrow_v, out_v,
              sp0, sp1, sq0, sq1):
    wid = lax.axis_index("s") * NC + lax.axis_index("c")
    base = wid * BPW

    pltpu.sync_copy(u_hbm.at[wid], uidx_v)
    pltpu.sync_copy(i_hbm.at[wid], iidx_v)

    one = jnp.full((L,), 1, jnp.int32)
    for c in range(NCHUNK):
        for g in range(CHUNK // L):
            sl = pl.ds(g * L, L)
            u = uidx_v[c, sl]
            uidx_v[c, sl] = lax.shift_right_logical(u, one)
            uoff_v[c, sl] = lax.shift_left(jnp.bitwise_and(u, one),
                                           jnp.full((L,), 6, jnp.int32))
            i = iidx_v[c, sl]
            iidx_v[c, sl] = lax.shift_right_logical(i, one)
            ioff_v[c, sl] = lax.shift_left(jnp.bitwise_and(i, one),
                                           jnp.full((L,), 6, jnp.int32))

    sems_p = (sp0, sp1)
    sems_q = (sq0, sq1)

    def fire(j):
        bsl = j % 2
        return (pltpu.async_copy(p_hbm.at[uidx_v.at[j]], prow_v.at[bsl], sems_p[bsl]),
                pltpu.async_copy(q_hbm.at[iidx_v.at[j]], qrow_v.at[bsl], sems_q[bsl]))

    lanes = lax.iota(jnp.int32, L)
    pending = fire(0)

    for j in range(NCHUNK):
        cur = pending
        if j + 1 < NCHUNK:
            nxt = fire(j + 1)
        cur[0].wait()
        cur[1].wait()
        if j + 1 < NCHUNK:
            pending = nxt
        bsl = j % 2

        def group(g, _, j=j, bsl=bsl):
            vec = jnp.zeros((L,), jnp.float32)
            uoffs = uoff_v[j, pl.ds(g * L, L)]
            ioffs = ioff_v[j, pl.ds(g * L, L)]
            for rr in range(L):
                r = g * L + rr
                po = uoffs[rr]
                qo = ioffs[rr]
                acc = prow_v[bsl, r, pl.ds(po, L)] * qrow_v[bsl, r, pl.ds(qo, L)]
                for k in range(1, D // L):
                    acc = acc + (prow_v[bsl, r, pl.ds(po + k * L, L)]
                                 * qrow_v[bsl, r, pl.ds(qo + k * L, L)])
                vec = jnp.where(lanes == rr, jnp.sum(acc), vec)
            out_v[pl.ds(j * CHUNK + g * L, L)] = vec
            return _

        lax.fori_loop(0, CHUNK // L, group, None)

    pltpu.sync_copy(out_v, out_hbm.at[pl.ds(base, BPW)])


_dot = functools.partial(
    pl.kernel,
    out_type=jax.ShapeDtypeStruct((B,), jnp.float32),
    compiler_params=pltpu.CompilerParams(
        needs_layout_passes=False, use_tc_tiling_on_sc=True),
    scratch_types=[
        pltpu.VMEM((NCHUNK, CHUNK), jnp.int32),
        pltpu.VMEM((NCHUNK, CHUNK), jnp.int32),
        pltpu.VMEM((NCHUNK, CHUNK), jnp.int32),
        pltpu.VMEM((NCHUNK, CHUNK), jnp.int32),
        pltpu.VMEM((2, CHUNK, DP), jnp.float32),
        pltpu.VMEM((2, CHUNK, DP), jnp.float32),
        pltpu.VMEM((BPW,), jnp.float32),
        pltpu.SemaphoreType.DMA,
        pltpu.SemaphoreType.DMA,
        pltpu.SemaphoreType.DMA,
        pltpu.SemaphoreType.DMA,
    ],
    **_SC_MESH,
)(_dot_body)


def _bias_body(u_hbm, i_hbm, bu_hbm, bi_hbm, out_hbm,
               uidx_v, iidx_v, buv_v, biv_v, sem):
    wid = lax.axis_index("s") * NC + lax.axis_index("c")
    base = wid * BPW

    pltpu.sync_copy(u_hbm.at[wid], uidx_v)
    pltpu.sync_copy(i_hbm.at[wid], iidx_v)

    copies = []
    for j in range(NCHUNK):
        sl = pl.ds(j * CHUNK, CHUNK)
        copies.append(pltpu.async_copy(bu_hbm.at[uidx_v.at[j]], buv_v.at[sl], sem))
        copies.append(pltpu.async_copy(bi_hbm.at[iidx_v.at[j]], biv_v.at[sl], sem))
    for c in copies:
        c.wait()

    for g in range(BPW // L):
        sl = pl.ds(g * L, L)
        buv_v[sl] = buv_v[sl] + biv_v[sl]

    pltpu.sync_copy(buv_v, out_hbm.at[pl.ds(base, BPW)])


_bias = functools.partial(
    pl.kernel,
    out_type=jax.ShapeDtypeStruct((B,), jnp.float32),
    compiler_params=pltpu.CompilerParams(
        needs_layout_passes=False, use_tc_tiling_on_sc=False),
    scratch_types=[
        pltpu.VMEM((NCHUNK, CHUNK), jnp.int32),
        pltpu.VMEM((NCHUNK, CHUNK), jnp.int32),
        pltpu.VMEM((BPW,), jnp.float32),
        pltpu.VMEM((BPW,), jnp.float32),
        pltpu.SemaphoreType.DMA,
    ],
    **_SC_MESH,
)(_bias_body)


@jax.jit
def kernel(u_idx, i_idx, mu, b_u, b_i, P, Q):
    u2 = u_idx.astype(jnp.int32).reshape(NW, NCHUNK, CHUNK)
    i2 = i_idx.astype(jnp.int32).reshape(NW, NCHUNK, CHUNK)
    p2 = P.reshape(P.shape[0] // 2, DP)
    q2 = Q.reshape(Q.shape[0] // 2, DP)
    dot = _dot(u2, i2, p2, q2)
    bias = _bias(u2, i2, b_u, b_i)
    return dot + bias + mu
